# Initial kernel scaffold; baseline (speedup 1.0000x reference)
import jax
import jax.numpy as jnp
from jax.experimental import pallas as pl


def kernel(x, edge_index, W_gnn, b_gnn, W1e, b1e, W2e, b2e, W1t, b1t, W2t, b2t):
    raise NotImplementedError("write your pallas kernel here")


# speedup vs baseline: 9.4797x; 9.4797x over previous
import functools

import jax
import jax.numpy as jnp
import numpy as np
from jax import lax
from jax.experimental import pallas as pl
from jax.experimental.pallas import tpu as pltpu
from jax.experimental.pallas import tpu_sc as plsc

N, E, D, H = 10000, 160000, 128, 16
NC, NS = 2, 16
NW = NC * NS
EPW = E // NW
EC1 = 200
NPT = N // NS
E8 = E // 8
ER = 1000
BN = 1000
NINF = np.float32(-np.inf)

_mesh = lambda: plsc.VectorSubcoreMesh(core_axis_name="c", subcore_axis_name="s")


def _sc1_body(src_hbm, dst_hbm, x_hbm, out_hbm, idx_s, idx_d, rows, agg_sh, sem):
    cid = lax.axis_index("c")
    sid = lax.axis_index("s")

    def zrow(i, c):
        for j in range(D // 16):
            rows[i, pl.ds(j * 16, 16)] = jnp.zeros((16,), jnp.float32)
        return c

    lax.fori_loop(0, EC1, zrow, 0)
    for off, size in ((0, 200), (200, 200), (400, 200), (600, 24)):
        pltpu.sync_copy(rows.at[pl.ds(0, size)],
                        agg_sh.at[pl.ds(sid * 624 + off, size)])

    @pl.when(sid == NS - 1)
    def _():
        pltpu.sync_copy(rows.at[pl.ds(0, 16)], agg_sh.at[pl.ds(624 * NS, 16)])

    plsc.subcore_barrier()

    base = cid * (E // NC) + sid * EPW

    def chunk(i, c):
        eb = pl.multiple_of(base + i * EC1, 8)
        pltpu.sync_copy(src_hbm.at[pl.ds(eb, EC1)], idx_s)
        pltpu.sync_copy(dst_hbm.at[pl.ds(eb, EC1)], idx_d)
        pltpu.async_copy(x_hbm.at[idx_s], rows, sem).wait()
        pltpu.sync_copy(rows, agg_sh.at[idx_d], add=True)
        return c

    lax.fori_loop(0, EPW // EC1, chunk, 0)
    plsc.subcore_barrier()

    b = sid * 624
    for off, size in ((0, 200), (200, 200), (400, 200), (600, 24)):
        pltpu.sync_copy(agg_sh.at[pl.ds(b + off, size)], rows.at[pl.ds(0, size)])
        pltpu.sync_copy(rows.at[pl.ds(0, size)],
                        out_hbm.at[pl.ds(cid * N + b + off, size)])

    @pl.when(sid == NS - 1)
    def _():
        t = 624 * NS
        pltpu.sync_copy(agg_sh.at[pl.ds(t, 16)], rows.at[pl.ds(0, 16)])
        pltpu.sync_copy(rows.at[pl.ds(0, 16)], out_hbm.at[pl.ds(cid * N + t, 16)])


def _sc1(src, dst, x):
    k = pl.kernel(
        _sc1_body,
        out_type=jax.ShapeDtypeStruct((NC * N, D), jnp.float32),
        mesh=_mesh(),
        scratch_types=[
            pltpu.VMEM((EC1,), jnp.int32),
            pltpu.VMEM((EC1,), jnp.int32),
            pltpu.VMEM((EC1, D), jnp.float32),
            pltpu.VMEM_SHARED((N, D), jnp.float32),
            pltpu.SemaphoreType.DMA,
        ],
    )
    return k(src, dst, x)


def _tc1_body(x_ref, a0_ref, a1_ref, wg_ref, bg_ref, whe_ref, wht_ref,
              emb_ref, nh_ref, nt_ref, gs_ref):
    xb = x_ref[...] + a0_ref[...] + a1_ref[...]
    emb = jnp.maximum(
        jnp.dot(xb, wg_ref[...], preferred_element_type=jnp.float32) + bg_ref[...],
        0.0)
    emb_ref[...] = emb
    nh_ref[...] = jnp.dot(emb, whe_ref[...], preferred_element_type=jnp.float32)
    nt_ref[...] = jnp.dot(emb, wht_ref[...], preferred_element_type=jnp.float32)
    s = jnp.sum(emb, axis=0, keepdims=True)

    @pl.when(pl.program_id(0) == 0)
    def _():
        gs_ref[...] = s

    @pl.when(pl.program_id(0) != 0)
    def _():
        gs_ref[...] = gs_ref[...] + s


def _tc1(x, agg2, W_gnn, b_gnn, W1e_bot, W1t_bot):
    nsteps = N // BN
    return pl.pallas_call(
        _tc1_body,
        grid=(nsteps,),
        in_specs=[
            pl.BlockSpec((BN, D), lambda i: (i, 0)),
            pl.BlockSpec((BN, D), lambda i: (i, 0)),
            pl.BlockSpec((BN, D), lambda i: (i + nsteps, 0)),
            pl.BlockSpec((D, D), lambda i: (0, 0)),
            pl.BlockSpec((1, D), lambda i: (0, 0)),
            pl.BlockSpec((D, H), lambda i: (0, 0)),
            pl.BlockSpec((D, H), lambda i: (0, 0)),
        ],
        out_specs=[
            pl.BlockSpec((BN, D), lambda i: (i, 0)),
            pl.BlockSpec((BN, H), lambda i: (i, 0)),
            pl.BlockSpec((BN, H), lambda i: (i, 0)),
            pl.BlockSpec((1, D), lambda i: (0, 0)),
        ],
        out_shape=[
            jax.ShapeDtypeStruct((N, D), jnp.float32),
            jax.ShapeDtypeStruct((N, H), jnp.float32),
            jax.ShapeDtypeStruct((N, H), jnp.float32),
            jax.ShapeDtypeStruct((1, D), jnp.float32),
        ],
    )(x, agg2, agg2, W_gnn, b_gnn, W1e_bot, W1t_bot)


def _sc2_body(src_hbm, dst_hbm, nh_hbm, outs_hbm, outd_hbm, idx, rows, sem):
    cid = lax.axis_index("c")
    sid = lax.axis_index("s")
    base = pl.multiple_of((cid * NS + sid) * EPW, 8)
    pltpu.sync_copy(src_hbm.at[pl.ds(base, EPW)], idx)
    pltpu.async_copy(nh_hbm.at[idx], rows, sem).wait()
    pltpu.sync_copy(rows, outs_hbm.at[pl.ds(base, EPW)])
    pltpu.sync_copy(dst_hbm.at[pl.ds(base, EPW)], idx)
    pltpu.async_copy(nh_hbm.at[idx], rows, sem).wait()
    pltpu.sync_copy(rows, outd_hbm.at[pl.ds(base, EPW)])


def _sc2(src, dst, nodeH):
    k = pl.kernel(
        _sc2_body,
        out_type=(jax.ShapeDtypeStruct((E, H), jnp.float32),
                  jax.ShapeDtypeStruct((E, H), jnp.float32)),
        mesh=_mesh(),
        compiler_params=pltpu.CompilerParams(use_tc_tiling_on_sc=False),
        scratch_types=[
            pltpu.VMEM((EPW,), jnp.int32),
            pltpu.VMEM((EPW, H), jnp.float32),
            pltpu.SemaphoreType.DMA,
        ],
    )
    return k(src, dst, nodeH)


def _tc2_body(hss, hsd, nz, sr, dr, gs, w1t, b1t, me, b2,
              eidx_o, vfir_o, vsec_o, lpe_o, Mr, Sr, Bv, Bl, Bi, Bs, Bd):
    step = pl.program_id(0)

    @pl.when(step == 0)
    def _():
        Mr[0] = NINF
        Sr[0] = jnp.float32(0.0)
        Bv[0] = NINF
        Bl[0] = jnp.float32(0.0)
        Bi[0] = jnp.int32(0)
        Bs[0] = jnp.int32(0)
        Bd[0] = jnp.int32(0)

    cE = jnp.dot(gs[...] / N, w1t[...], preferred_element_type=jnp.float32) + b1t[...]
    sg = jax.nn.sigmoid(hss[...] + hsd[...] + cE)
    lg = jnp.dot(sg, me[...], preferred_element_type=jnp.float32) + b2[...]
    y = lg + nz[...]
    rid = lax.broadcasted_iota(jnp.int32, (ER, 8), 0)
    cid = lax.broadcasted_iota(jnp.int32, (ER, 8), 1)
    ids = (step * ER + rid) * 8 + cid

    bm = jnp.max(y)
    bi = jnp.min(jnp.where(y == bm, ids, jnp.int32(2**31 - 1)))
    selm = ids == bi
    wl = jnp.max(jnp.where(selm, lg, NINF))
    ws = jnp.max(jnp.where(selm, sr[...], jnp.int32(-1)))
    wd = jnp.max(jnp.where(selm, dr[...], jnp.int32(-1)))

    lmax = jnp.max(lg)
    newM = jnp.maximum(Mr[0], lmax)
    Sr[0] = Sr[0] * jnp.exp(Mr[0] - newM) + jnp.sum(jnp.exp(lg - newM))
    Mr[0] = newM

    upd = bm > Bv[0]
    Bv[0] = jnp.where(upd, bm, Bv[0])
    Bi[0] = jnp.where(upd, bi, Bi[0])
    Bl[0] = jnp.where(upd, wl, Bl[0])
    Bs[0] = jnp.where(upd, ws, Bs[0])
    Bd[0] = jnp.where(upd, wd, Bd[0])

    eidx_o[0] = Bi[0]
    vfir_o[0] = Bd[0]
    vsec_o[0] = Bs[0]
    lpe_o[0] = Bl[0] - Mr[0] - jnp.log(Sr[0])


def _tc2(hs_s, hs_d, noise8, src8, dst8, gsum, W1e_top_t, b1e_t, Me, b2e_t):
    return pl.pallas_call(
        _tc2_body,
        grid=(E8 // ER,),
        in_specs=[
            pl.BlockSpec((ER, D), lambda i: (i, 0)),
            pl.BlockSpec((ER, D), lambda i: (i, 0)),
            pl.BlockSpec((ER, 8), lambda i: (i, 0)),
            pl.BlockSpec((ER, 8), lambda i: (i, 0)),
            pl.BlockSpec((ER, 8), lambda i: (i, 0)),
            pl.BlockSpec((1, D), lambda i: (0, 0)),
            pl.BlockSpec((D, D), lambda i: (0, 0)),
            pl.BlockSpec((1, D), lambda i: (0, 0)),
            pl.BlockSpec((D, 8), lambda i: (0, 0)),
            pl.BlockSpec((1, 8), lambda i: (0, 0)),
        ],
        out_specs=[
            pl.BlockSpec(memory_space=pltpu.SMEM),
            pl.BlockSpec(memory_space=pltpu.SMEM),
            pl.BlockSpec(memory_space=pltpu.SMEM),
            pl.BlockSpec(memory_space=pltpu.SMEM),
        ],
        out_shape=[
            jax.ShapeDtypeStruct((1,), jnp.int32),
            jax.ShapeDtypeStruct((1,), jnp.int32),
            jax.ShapeDtypeStruct((1,), jnp.int32),
            jax.ShapeDtypeStruct((1,), jnp.float32),
        ],
        scratch_shapes=[
            pltpu.SMEM((1,), jnp.float32),
            pltpu.SMEM((1,), jnp.float32),
            pltpu.SMEM((1,), jnp.float32),
            pltpu.SMEM((1,), jnp.float32),
            pltpu.SMEM((1,), jnp.int32),
            pltpu.SMEM((1,), jnp.int32),
            pltpu.SMEM((1,), jnp.int32),
        ],
    )(hs_s, hs_d, noise8, src8, dst8, gsum, W1e_top_t, b1e_t, Me, b2e_t)


def _tc3_body(ntr, emb, gs, vf_s, vs_s, wgt, wst, wft, b1t_t, mt, b2t_t, out):
    vf = vf_s[0]
    vs = vs_s[0]
    ef = emb[pl.ds(vf, 1), :]
    es = emb[pl.ds(vs, 1), :]
    c3 = (jnp.dot(gs[...] / N, wgt[...], preferred_element_type=jnp.float32)
          + jnp.dot(ef + es, wst[...], preferred_element_type=jnp.float32)
          + jnp.dot(ef, wft[...], preferred_element_type=jnp.float32)
          + b1t_t[...])
    sg = jax.nn.sigmoid(ntr[...] + c3)
    out[...] = jnp.dot(sg, mt[...], preferred_element_type=jnp.float32) + b2t_t[...]


def _tc3(ntr, emb, gsum, vfir, vsec, Wg_t, Ws_t, Wf_t, b1t_t, Mt, b2t_t):
    return pl.pallas_call(
        _tc3_body,
        in_specs=[
            pl.BlockSpec((N // 8, D), lambda: (0, 0)),
            pl.BlockSpec((N, D), lambda: (0, 0)),
            pl.BlockSpec((1, D), lambda: (0, 0)),
            pl.BlockSpec(memory_space=pltpu.SMEM),
            pl.BlockSpec(memory_space=pltpu.SMEM),
            pl.BlockSpec((D, D), lambda: (0, 0)),
            pl.BlockSpec((D, D), lambda: (0, 0)),
            pl.BlockSpec((D, D), lambda: (0, 0)),
            pl.BlockSpec((1, D), lambda: (0, 0)),
            pl.BlockSpec((D, 8), lambda: (0, 0)),
            pl.BlockSpec((1, 8), lambda: (0, 0)),
        ],
        out_specs=pl.BlockSpec((N // 8, 8), lambda: (0, 0)),
        out_shape=jax.ShapeDtypeStruct((N // 8, 8), jnp.float32),
    )(ntr, emb, gsum, vfir, vsec, Wg_t, Ws_t, Wf_t, b1t_t, Mt, b2t_t)


def _sc3_body(src_hbm, dst_hbm, vf_hbm, lg_hbm, gn_hbm, resi_hbm, resf_hbm,
              dstb, srcb, valb, cnt_v, lg_v, gn_v, vf_v, v16, i16b,
              resv_i, resv_f, cnt_sh, sem):
    cid = lax.axis_index("c")
    sid = lax.axis_index("s")
    on0 = cid == 0
    lanes = lax.iota(jnp.int32, 16)

    @pl.when(jnp.logical_and(on0, sid == 0))
    def _():
        def z(i, c):
            valb[pl.ds(i * 16, 16)] = jnp.zeros((16,), jnp.int32)
            return c
        lax.fori_loop(0, N // 16, z, 0)
        pltpu.sync_copy(valb.at[pl.ds(0, N)], cnt_sh)

    plsc.subcore_barrier()

    @pl.when(on0)
    def _():
        pltpu.sync_copy(vf_hbm, vf_v)
        vf = vf_v[...]
        eb = pl.multiple_of(sid * (E // NS), 8)
        pltpu.sync_copy(dst_hbm.at[pl.ds(eb, E // NS)], dstb)
        pltpu.sync_copy(src_hbm.at[pl.ds(eb, E // NS)], srcb)

        def cmp(i, c):
            d = dstb[pl.ds(i * 16, 16)]
            valb[pl.ds(i * 16, 16)] = jnp.where(d == vf, 1, 0).astype(jnp.int32)
            return c

        lax.fori_loop(0, (E // NS) // 16, cmp, 0)
        pltpu.sync_copy(valb, cnt_sh.at[srcb], add=True)

        @pl.when(sid == 0)
        def _():
            v16[...] = jnp.where(lanes == 0, 1, 0).astype(jnp.int32)
            i16b[...] = vf
            pltpu.sync_copy(v16, cnt_sh.at[i16b], add=True)

    plsc.subcore_barrier()

    @pl.when(jnp.logical_and(on0, sid == 0))
    def _():
        pltpu.sync_copy(cnt_sh, cnt_v)
        pltpu.sync_copy(lg_hbm, lg_v)
        pltpu.sync_copy(gn_hbm, gn_v)

        def pA(i, m):
            l = lg_v[pl.ds(i * 16, 16)]
            c = cnt_v[pl.ds(i * 16, 16)]
            return jnp.maximum(m, jnp.where(c > 0, NINF, l))

        mreg = lax.fori_loop(0, N // 16, pA, jnp.full((16,), NINF, jnp.float32))
        mxs = jnp.max(mreg, axis=0)
        mxv = jnp.broadcast_to(mxs, (16,))

        def pB(i, carry):
            cp, se, bv, bn, bl = carry
            l = lg_v[pl.ds(i * 16, 16)]
            c = cnt_v[pl.ds(i * 16, 16)]
            validb = c == 0
            vi = jnp.where(validb, 1, 0).astype(jnp.int32)
            incl = plsc.cumsum(vi)
            pos = cp + incl - vi
            gn = plsc.load_gather(gn_v, [pos])
            val = jnp.where(validb, l + gn, NINF)
            upd = val > bv
            bv = jnp.where(upd, val, bv)
            bn = jnp.where(upd, lanes + i * 16, bn)
            bl = jnp.where(upd, l, bl)
            se = se + jnp.where(validb, jnp.exp(l - mxv), jnp.float32(0.0))
            cp = cp + plsc.all_reduce_population_count(validb)
            return (cp, se, bv, bn, bl)

        z16i = jnp.zeros((16,), jnp.int32)
        cp, se, bv, bn, bl = lax.fori_loop(
            0, N // 16, pB,
            (z16i, jnp.zeros((16,), jnp.float32), jnp.full((16,), NINF, jnp.float32),
             z16i, jnp.zeros((16,), jnp.float32)))

        M = jnp.max(bv, axis=0)
        eq = bv == jnp.broadcast_to(M, (16,))
        wn = jnp.min(jnp.where(eq, bn, jnp.int32(2**31 - 1)), axis=0)
        wnv = jnp.broadcast_to(wn, (16,))
        blw = jnp.max(jnp.where(jnp.logical_and(eq, bn == wnv), bl, NINF), axis=0)
        S = jnp.sum(se, axis=0)

        resv_i[...] = wnv
        resv_f[...] = jnp.where(lanes == 0, jnp.broadcast_to(blw, (16,)),
                                jnp.where(lanes == 1, mxv, jnp.broadcast_to(S, (16,))))
        pltpu.sync_copy(resv_i, resi_hbm)
        pltpu.sync_copy(resv_f, resf_hbm)


def _sc3(src, dst, vf16, lg, gn):
    k = pl.kernel(
        _sc3_body,
        out_type=(jax.ShapeDtypeStruct((16,), jnp.int32),
                  jax.ShapeDtypeStruct((16,), jnp.float32)),
        mesh=_mesh(),
        compiler_params=pltpu.CompilerParams(needs_layout_passes=False),
        scratch_types=[
            pltpu.VMEM((E // NS,), jnp.int32),
            pltpu.VMEM((E // NS,), jnp.int32),
            pltpu.VMEM((E // NS,), jnp.int32),
            pltpu.VMEM((N,), jnp.int32),
            pltpu.VMEM((N,), jnp.float32),
            pltpu.VMEM((N,), jnp.float32),
            pltpu.VMEM((16,), jnp.int32),
            pltpu.VMEM((16,), jnp.int32),
            pltpu.VMEM((16,), jnp.int32),
            pltpu.VMEM((16,), jnp.int32),
            pltpu.VMEM((16,), jnp.float32),
            pltpu.VMEM_SHARED((N,), jnp.int32),
            pltpu.SemaphoreType.DMA,
        ],
    )
    return k(src, dst, vf16, lg, gn)


def kernel(x, edge_index, W_gnn, b_gnn, W1e, b1e, W2e, b2e, W1t, b1t, W2t, b2t):
    src = edge_index[0]
    dst = edge_index[1]

    noiseE = jax.random.gumbel(jax.random.key(42), (E,), jnp.float32)
    gnoise = jax.random.gumbel(jax.random.key(43), (N,), jnp.float32)

    agg2 = _sc1(src, dst, x)

    emb, nodeH, nodeT, gsum = _tc1(x, agg2, W_gnn, b_gnn[None, :],
                                   W1e[D:], W1t[3 * D:])

    hs_s, hs_d = _sc2(src, dst, nodeH)

    eye8 = jnp.eye(8, dtype=jnp.float32)
    W1e_top_t = jnp.tile(W1e[:D], (1, 8))
    b1e_t = jnp.tile(b1e, 8)[None, :]
    Me = jnp.kron(eye8, W2e)
    b2e_t = jnp.full((1, 8), b2e[0], jnp.float32)

    eidx, vfir, vsec, lpe = _tc2(
        hs_s.reshape(E8, D), hs_d.reshape(E8, D), noiseE.reshape(E8, 8),
        src.reshape(E8, 8), dst.reshape(E8, 8), gsum,
        W1e_top_t, b1e_t, Me, b2e_t)

    Wg_t = jnp.tile(W1t[:D], (1, 8))
    Ws_t = jnp.tile(W1t[D:2 * D], (1, 8))
    Wf_t = jnp.tile(W1t[2 * D:3 * D], (1, 8))
    b1t_t = jnp.tile(b1t, 8)[None, :]
    Mt = jnp.kron(eye8, W2t)
    b2t_t = jnp.full((1, 8), b2t[0], jnp.float32)

    tl8 = _tc3(nodeT.reshape(N // 8, D), emb, gsum,
               vfir.reshape(1), vsec.reshape(1),
               Wg_t, Ws_t, Wf_t, b1t_t, Mt, b2t_t)

    vf16 = jnp.broadcast_to(vfir.reshape(1), (16,)).astype(jnp.int32)
    resi, resf = _sc3(src, dst, vf16, tl8.reshape(N), gnoise)

    v_thi = resi[0]
    lp3 = (resf[0] - resf[1]) - jnp.log(resf[2])

    action = jnp.stack([vfir[0].astype(jnp.int32),
                        vsec[0].astype(jnp.int32),
                        v_thi.astype(jnp.int32)])
    return action, (lpe[0] + lp3).astype(jnp.float32)


# speedup vs baseline: 12.9741x; 1.3686x over previous
import functools

import jax
import jax.numpy as jnp
import numpy as np
from jax import lax
from jax.experimental import pallas as pl
from jax.experimental.pallas import tpu as pltpu
from jax.experimental.pallas import tpu_sc as plsc

N, E, D, H = 10000, 160000, 128, 16
NC, NS = 2, 16
NW = NC * NS
EPW = E // NW
EC1 = 152
EC1T = 136
NPT = N // NS
E8 = E // 8
ER = 2000
BN = 2000
NINF = np.float32(-np.inf)

_mesh = lambda: plsc.VectorSubcoreMesh(core_axis_name="c", subcore_axis_name="s")


def _sc1_body(src_hbm, dst_hbm, x_hbm, out_hbm, idx_s, idx_d, rows0, rows1,
              agg_sh, sg0, sg1, ss0, ss1):
    cid = lax.axis_index("c")
    sid = lax.axis_index("s")
    base = pl.multiple_of(cid * (E // NC) + sid * EPW, 8)

    pltpu.sync_copy(src_hbm.at[pl.ds(base, EPW)], idx_s)
    pltpu.sync_copy(dst_hbm.at[pl.ds(base, EPW)], idx_d)

    def gidx(c, size=EC1):
        return idx_s.at[pl.ds(c * EC1, size)]

    def didx(c, size=EC1):
        return idx_d.at[pl.ds(c * EC1, size)]

    def zrow(i, c):
        for j in range(D // 16):
            rows0[i, pl.ds(j * 16, 16)] = jnp.zeros((16,), jnp.float32)
        return c

    lax.fori_loop(0, EC1, zrow, 0)
    for off, size in ((0, 152), (152, 152), (304, 152), (456, 152), (608, 16)):
        pltpu.sync_copy(rows0.at[pl.ds(0, size)],
                        agg_sh.at[pl.ds(sid * 624 + off, size)])

    @pl.when(sid == NS - 1)
    def _():
        pltpu.sync_copy(rows0.at[pl.ds(0, 16)], agg_sh.at[pl.ds(624 * NS, 16)])

    plsc.subcore_barrier()

    pltpu.async_copy(x_hbm.at[gidx(0)], rows0, sg0)

    def body(j, c):
        c0 = 2 * j
        pltpu.make_async_copy(x_hbm.at[gidx(c0)], rows0, sg0).wait()
        s0 = pltpu.async_copy(rows0, agg_sh.at[didx(c0)], ss0, add=True)

        @pl.when(j > 0)
        def _():
            pltpu.make_async_copy(rows1, agg_sh.at[didx(c0 - 1)], ss1).wait()

        pltpu.async_copy(x_hbm.at[gidx(c0 + 1)], rows1, sg1)
        pltpu.make_async_copy(x_hbm.at[gidx(c0 + 1)], rows1, sg1).wait()
        pltpu.async_copy(rows1, agg_sh.at[didx(c0 + 1)], ss1, add=True)
        s0.wait()

        @pl.when(j < (EPW // EC1) // 2 - 1)
        def _():
            pltpu.async_copy(x_hbm.at[gidx(c0 + 2)], rows0, sg0)

        return c

    lax.fori_loop(0, (EPW // EC1) // 2, body, 0)
    pltpu.make_async_copy(rows1, agg_sh.at[didx(EPW // EC1 - 1)], ss1).wait()

    nc = EPW // EC1
    pltpu.async_copy(x_hbm.at[gidx(nc, EC1T)], rows0.at[pl.ds(0, EC1T)], sg0).wait()
    pltpu.async_copy(rows0.at[pl.ds(0, EC1T)], agg_sh.at[didx(nc, EC1T)],
                     ss0, add=True).wait()

    plsc.subcore_barrier()

    b = sid * 624
    for off, size in ((0, 152), (152, 152), (304, 152), (456, 152), (608, 16)):
        pltpu.sync_copy(agg_sh.at[pl.ds(b + off, size)], rows0.at[pl.ds(0, size)])
        pltpu.sync_copy(rows0.at[pl.ds(0, size)],
                        out_hbm.at[pl.ds(cid * N + b + off, size)])

    @pl.when(sid == NS - 1)
    def _():
        t = 624 * NS
        pltpu.sync_copy(agg_sh.at[pl.ds(t, 16)], rows0.at[pl.ds(0, 16)])
        pltpu.sync_copy(rows0.at[pl.ds(0, 16)], out_hbm.at[pl.ds(cid * N + t, 16)])


def _sc1(src, dst, x):
    k = pl.kernel(
        _sc1_body,
        out_type=jax.ShapeDtypeStruct((NC * N, D), jnp.float32),
        mesh=_mesh(),
        compiler_params=pltpu.CompilerParams(use_tc_tiling_on_sc=False),
        scratch_types=[
            pltpu.VMEM((EPW,), jnp.int32),
            pltpu.VMEM((EPW,), jnp.int32),
            pltpu.VMEM((EC1, D), jnp.float32),
            pltpu.VMEM((EC1, D), jnp.float32),
            pltpu.VMEM_SHARED((N, D), jnp.float32),
            pltpu.SemaphoreType.DMA,
            pltpu.SemaphoreType.DMA,
            pltpu.SemaphoreType.DMA,
            pltpu.SemaphoreType.DMA,
        ],
    )
    return k(src, dst, x)


def _tc1_body(x_ref, a0_ref, a1_ref, wg_ref, bg_ref, whe_ref, wht_ref, nz_ref,
              emb_ref, nh_ref, nt_ref, gs_ref, nzo_ref):
    nzo_ref[...] = nz_ref[...]
    xb = x_ref[...] + a0_ref[...] + a1_ref[...]
    emb = jnp.maximum(
        jnp.dot(xb, wg_ref[...], preferred_element_type=jnp.float32) + bg_ref[...],
        0.0)
    emb_ref[...] = emb
    nh_ref[...] = jnp.dot(emb, whe_ref[...], preferred_element_type=jnp.float32)
    nt_ref[...] = jnp.dot(emb, wht_ref[...], preferred_element_type=jnp.float32)
    s = jnp.sum(emb, axis=0, keepdims=True)

    @pl.when(pl.program_id(0) == 0)
    def _():
        gs_ref[...] = s

    @pl.when(pl.program_id(0) != 0)
    def _():
        gs_ref[...] = gs_ref[...] + s


def _tc1(x, agg2, W_gnn, b_gnn, W1e_bot, W1t_bot, noise8):
    nsteps = N // BN
    nzr = E8 // nsteps
    return pl.pallas_call(
        _tc1_body,
        grid=(nsteps,),
        in_specs=[
            pl.BlockSpec((BN, D), lambda i: (i, 0)),
            pl.BlockSpec((BN, D), lambda i: (i, 0)),
            pl.BlockSpec((BN, D), lambda i: (i + nsteps, 0)),
            pl.BlockSpec((D, D), lambda i: (0, 0)),
            pl.BlockSpec((1, D), lambda i: (0, 0)),
            pl.BlockSpec((D, H), lambda i: (0, 0)),
            pl.BlockSpec((D, H), lambda i: (0, 0)),
            pl.BlockSpec((nzr, 8), lambda i: (i, 0)),
        ],
        out_specs=[
            pl.BlockSpec((BN, D), lambda i: (i, 0)),
            pl.BlockSpec((BN, H), lambda i: (i, 0)),
            pl.BlockSpec((BN, H), lambda i: (i, 0)),
            pl.BlockSpec((1, D), lambda i: (0, 0)),
            pl.BlockSpec((nzr, 8), lambda i: (i, 0)),
        ],
        out_shape=[
            jax.ShapeDtypeStruct((N, D), jnp.float32),
            jax.ShapeDtypeStruct((N, H), jnp.float32),
            jax.ShapeDtypeStruct((N, H), jnp.float32),
            jax.ShapeDtypeStruct((1, D), jnp.float32),
            jax.ShapeDtypeStruct((E8, 8), jnp.float32),
        ],
    )(x, agg2, agg2, W_gnn, b_gnn, W1e_bot, W1t_bot, noise8)


def _sc2_body(src_hbm, dst_hbm, nh_hbm, outs_hbm, outd_hbm, idx, rows, sem):
    cid = lax.axis_index("c")
    sid = lax.axis_index("s")
    base = pl.multiple_of((cid * NS + sid) * EPW, 8)
    pltpu.sync_copy(src_hbm.at[pl.ds(base, EPW)], idx)
    pltpu.async_copy(nh_hbm.at[idx], rows, sem).wait()
    pltpu.sync_copy(rows, outs_hbm.at[pl.ds(base, EPW)])
    pltpu.sync_copy(dst_hbm.at[pl.ds(base, EPW)], idx)
    pltpu.async_copy(nh_hbm.at[idx], rows, sem).wait()
    pltpu.sync_copy(rows, outd_hbm.at[pl.ds(base, EPW)])


def _sc2(src, dst, nodeH):
    k = pl.kernel(
        _sc2_body,
        out_type=(jax.ShapeDtypeStruct((E, H), jnp.float32),
                  jax.ShapeDtypeStruct((E, H), jnp.float32)),
        mesh=_mesh(),
        compiler_params=pltpu.CompilerParams(use_tc_tiling_on_sc=False),
        scratch_types=[
            pltpu.VMEM((EPW,), jnp.int32),
            pltpu.VMEM((EPW, H), jnp.float32),
            pltpu.SemaphoreType.DMA,
        ],
    )
    return k(src, dst, nodeH)


def _tc2_body(hss, hsd, nz, sr, dr, gs, w1t, b1t, me, b2,
              eidx_o, vfir_o, vsec_o, lpe_o, Mr, Sr, Bv, Bl, Bi, Bs, Bd):
    step = pl.program_id(0)

    @pl.when(step == 0)
    def _():
        Mr[0] = NINF
        Sr[0] = jnp.float32(0.0)
        Bv[0] = NINF
        Bl[0] = jnp.float32(0.0)
        Bi[0] = jnp.int32(0)
        Bs[0] = jnp.int32(0)
        Bd[0] = jnp.int32(0)

    cE = jnp.dot(gs[...] / N, w1t[...], preferred_element_type=jnp.float32) + b1t[...]
    sg = jax.nn.sigmoid(hss[...] + hsd[...] + cE)
    lg = jnp.dot(sg, me[...], preferred_element_type=jnp.float32) + b2[...]
    y = lg + nz[...]
    rid = lax.broadcasted_iota(jnp.int32, (ER, 8), 0)
    cid = lax.broadcasted_iota(jnp.int32, (ER, 8), 1)
    ids = (step * ER + rid) * 8 + cid

    bm = jnp.max(y)
    bi = jnp.min(jnp.where(y == bm, ids, jnp.int32(2**31 - 1)))
    selm = ids == bi
    wl = jnp.max(jnp.where(selm, lg, NINF))
    ws = jnp.max(jnp.where(selm, sr[...], jnp.int32(-1)))
    wd = jnp.max(jnp.where(selm, dr[...], jnp.int32(-1)))

    lmax = jnp.max(lg)
    newM = jnp.maximum(Mr[0], lmax)
    Sr[0] = Sr[0] * jnp.exp(Mr[0] - newM) + jnp.sum(jnp.exp(lg - newM))
    Mr[0] = newM

    upd = bm > Bv[0]
    Bv[0] = jnp.where(upd, bm, Bv[0])
    Bi[0] = jnp.where(upd, bi, Bi[0])
    Bl[0] = jnp.where(upd, wl, Bl[0])
    Bs[0] = jnp.where(upd, ws, Bs[0])
    Bd[0] = jnp.where(upd, wd, Bd[0])

    eidx_o[0] = Bi[0]
    vfir_o[0] = Bd[0]
    vsec_o[0] = Bs[0]
    lpe_o[0] = Bl[0] - Mr[0] - jnp.log(Sr[0])


def _tc2(hs_s, hs_d, noise8, src8, dst8, gsum, W1e_top_t, b1e_t, Me, b2e_t):
    return pl.pallas_call(
        _tc2_body,
        grid=(E8 // ER,),
        in_specs=[
            pl.BlockSpec((ER, D), lambda i: (i, 0)),
            pl.BlockSpec((ER, D), lambda i: (i, 0)),
            pl.BlockSpec((ER, 8), lambda i: (i, 0)),
            pl.BlockSpec((ER, 8), lambda i: (i, 0)),
            pl.BlockSpec((ER, 8), lambda i: (i, 0)),
            pl.BlockSpec((1, D), lambda i: (0, 0)),
            pl.BlockSpec((D, D), lambda i: (0, 0)),
            pl.BlockSpec((1, D), lambda i: (0, 0)),
            pl.BlockSpec((D, 8), lambda i: (0, 0)),
            pl.BlockSpec((1, 8), lambda i: (0, 0)),
        ],
        out_specs=[
            pl.BlockSpec(memory_space=pltpu.SMEM),
            pl.BlockSpec(memory_space=pltpu.SMEM),
            pl.BlockSpec(memory_space=pltpu.SMEM),
            pl.BlockSpec(memory_space=pltpu.SMEM),
        ],
        out_shape=[
            jax.ShapeDtypeStruct((1,), jnp.int32),
            jax.ShapeDtypeStruct((1,), jnp.int32),
            jax.ShapeDtypeStruct((1,), jnp.int32),
            jax.ShapeDtypeStruct((1,), jnp.float32),
        ],
        scratch_shapes=[
            pltpu.SMEM((1,), jnp.float32),
            pltpu.SMEM((1,), jnp.float32),
            pltpu.SMEM((1,), jnp.float32),
            pltpu.SMEM((1,), jnp.float32),
            pltpu.SMEM((1,), jnp.int32),
            pltpu.SMEM((1,), jnp.int32),
            pltpu.SMEM((1,), jnp.int32),
        ],
    )(hs_s, hs_d, noise8, src8, dst8, gsum, W1e_top_t, b1e_t, Me, b2e_t)


def _tc3_body(ntr, emb, gs, vf_s, vs_s, wgt, wst, wft, b1t_t, mt, b2t_t, out):
    vf = vf_s[0]
    vs = vs_s[0]
    ef = emb[pl.ds(vf, 1), :]
    es = emb[pl.ds(vs, 1), :]
    c3 = (jnp.dot(gs[...] / N, wgt[...], preferred_element_type=jnp.float32)
          + jnp.dot(ef + es, wst[...], preferred_element_type=jnp.float32)
          + jnp.dot(ef, wft[...], preferred_element_type=jnp.float32)
          + b1t_t[...])
    sg = jax.nn.sigmoid(ntr[...] + c3)
    out[...] = jnp.dot(sg, mt[...], preferred_element_type=jnp.float32) + b2t_t[...]


def _tc3(ntr, emb, gsum, vfir, vsec, Wg_t, Ws_t, Wf_t, b1t_t, Mt, b2t_t):
    return pl.pallas_call(
        _tc3_body,
        in_specs=[
            pl.BlockSpec((N // 8, D), lambda: (0, 0)),
            pl.BlockSpec((N, D), lambda: (0, 0)),
            pl.BlockSpec((1, D), lambda: (0, 0)),
            pl.BlockSpec(memory_space=pltpu.SMEM),
            pl.BlockSpec(memory_space=pltpu.SMEM),
            pl.BlockSpec((D, D), lambda: (0, 0)),
            pl.BlockSpec((D, D), lambda: (0, 0)),
            pl.BlockSpec((D, D), lambda: (0, 0)),
            pl.BlockSpec((1, D), lambda: (0, 0)),
            pl.BlockSpec((D, 8), lambda: (0, 0)),
            pl.BlockSpec((1, 8), lambda: (0, 0)),
        ],
        out_specs=pl.BlockSpec((N // 8, 8), lambda: (0, 0)),
        out_shape=jax.ShapeDtypeStruct((N // 8, 8), jnp.float32),
    )(ntr, emb, gsum, vfir, vsec, Wg_t, Ws_t, Wf_t, b1t_t, Mt, b2t_t)


def _sc3_body(src_hbm, dst_hbm, vf_hbm, lg_hbm, gn_hbm, resi_hbm, resf_hbm,
              dstb, srcb, valb, cnt_v, lg_v, gn_v, vf_v, v16, i16b,
              resv_i, resv_f, cnt_sh, sem):
    cid = lax.axis_index("c")
    sid = lax.axis_index("s")
    on0 = cid == 0
    lanes = lax.iota(jnp.int32, 16)

    @pl.when(jnp.logical_and(on0, sid == 0))
    def _():
        def z(i, c):
            valb[pl.ds(i * 16, 16)] = jnp.zeros((16,), jnp.int32)
            return c
        lax.fori_loop(0, N // 16, z, 0)
        pltpu.sync_copy(valb.at[pl.ds(0, N)], cnt_sh)

    plsc.subcore_barrier()

    @pl.when(on0)
    def _():
        pltpu.sync_copy(vf_hbm, vf_v)
        vf = vf_v[...]
        eb = pl.multiple_of(sid * (E // NS), 8)
        pltpu.sync_copy(dst_hbm.at[pl.ds(eb, E // NS)], dstb)
        pltpu.sync_copy(src_hbm.at[pl.ds(eb, E // NS)], srcb)

        def cmp(i, c):
            d = dstb[pl.ds(i * 16, 16)]
            valb[pl.ds(i * 16, 16)] = jnp.where(d == vf, 1, 0).astype(jnp.int32)
            return c

        lax.fori_loop(0, (E // NS) // 16, cmp, 0)
        pltpu.sync_copy(valb, cnt_sh.at[srcb], add=True)

        @pl.when(sid == 0)
        def _():
            v16[...] = jnp.where(lanes == 0, 1, 0).astype(jnp.int32)
            i16b[...] = vf
            pltpu.sync_copy(v16, cnt_sh.at[i16b], add=True)

    plsc.subcore_barrier()

    @pl.when(jnp.logical_and(on0, sid == 0))
    def _():
        pltpu.sync_copy(cnt_sh, cnt_v)
        pltpu.sync_copy(lg_hbm, lg_v)
        pltpu.sync_copy(gn_hbm, gn_v)

        def pA(i, m):
            l = lg_v[pl.ds(i * 16, 16)]
            c = cnt_v[pl.ds(i * 16, 16)]
            return jnp.maximum(m, jnp.where(c > 0, NINF, l))

        mreg = lax.fori_loop(0, N // 16, pA, jnp.full((16,), NINF, jnp.float32))
        mxs = jnp.max(mreg, axis=0)
        mxv = jnp.broadcast_to(mxs, (16,))

        def pB(i, carry):
            cp, se, bv, bn, bl = carry
            l = lg_v[pl.ds(i * 16, 16)]
            c = cnt_v[pl.ds(i * 16, 16)]
            validb = c == 0
            vi = jnp.where(validb, 1, 0).astype(jnp.int32)
            incl = plsc.cumsum(vi)
            pos = cp + incl - vi
            gn = plsc.load_gather(gn_v, [pos])
            val = jnp.where(validb, l + gn, NINF)
            upd = val > bv
            bv = jnp.where(upd, val, bv)
            bn = jnp.where(upd, lanes + i * 16, bn)
            bl = jnp.where(upd, l, bl)
            se = se + jnp.where(validb, jnp.exp(l - mxv), jnp.float32(0.0))
            cp = cp + plsc.all_reduce_population_count(validb)
            return (cp, se, bv, bn, bl)

        z16i = jnp.zeros((16,), jnp.int32)
        cp, se, bv, bn, bl = lax.fori_loop(
            0, N // 16, pB,
            (z16i, jnp.zeros((16,), jnp.float32), jnp.full((16,), NINF, jnp.float32),
             z16i, jnp.zeros((16,), jnp.float32)))

        M = jnp.max(bv, axis=0)
        eq = bv == jnp.broadcast_to(M, (16,))
        wn = jnp.min(jnp.where(eq, bn, jnp.int32(2**31 - 1)), axis=0)
        wnv = jnp.broadcast_to(wn, (16,))
        blw = jnp.max(jnp.where(jnp.logical_and(eq, bn == wnv), bl, NINF), axis=0)
        S = jnp.sum(se, axis=0)

        resv_i[...] = wnv
        resv_f[...] = jnp.where(lanes == 0, jnp.broadcast_to(blw, (16,)),
                                jnp.where(lanes == 1, mxv, jnp.broadcast_to(S, (16,))))
        pltpu.sync_copy(resv_i, resi_hbm)
        pltpu.sync_copy(resv_f, resf_hbm)


def _sc3(src, dst, vf16, lg, gn):
    k = pl.kernel(
        _sc3_body,
        out_type=(jax.ShapeDtypeStruct((16,), jnp.int32),
                  jax.ShapeDtypeStruct((16,), jnp.float32)),
        mesh=_mesh(),
        compiler_params=pltpu.CompilerParams(needs_layout_passes=False),
        scratch_types=[
            pltpu.VMEM((E // NS,), jnp.int32),
            pltpu.VMEM((E // NS,), jnp.int32),
            pltpu.VMEM((E // NS,), jnp.int32),
            pltpu.VMEM((N,), jnp.int32),
            pltpu.VMEM((N,), jnp.float32),
            pltpu.VMEM((N,), jnp.float32),
            pltpu.VMEM((16,), jnp.int32),
            pltpu.VMEM((16,), jnp.int32),
            pltpu.VMEM((16,), jnp.int32),
            pltpu.VMEM((16,), jnp.int32),
            pltpu.VMEM((16,), jnp.float32),
            pltpu.VMEM_SHARED((N,), jnp.int32),
            pltpu.SemaphoreType.DMA,
        ],
    )
    return k(src, dst, vf16, lg, gn)


def kernel(x, edge_index, W_gnn, b_gnn, W1e, b1e, W2e, b2e, W1t, b1t, W2t, b2t):
    src = edge_index[0]
    dst = edge_index[1]

    noiseE = jax.random.gumbel(jax.random.key(42), (E,), jnp.float32)
    gnoise = jax.random.gumbel(jax.random.key(43), (N,), jnp.float32)

    agg2 = _sc1(src, dst, x)

    emb, nodeH, nodeT, gsum, noise8 = _tc1(x, agg2, W_gnn, b_gnn[None, :],
                                           W1e[D:], W1t[3 * D:],
                                           noiseE.reshape(E8, 8))

    hs_s, hs_d = _sc2(src, dst, nodeH)

    eye8 = jnp.eye(8, dtype=jnp.float32)
    W1e_top_t = jnp.tile(W1e[:D], (1, 8))
    b1e_t = jnp.tile(b1e, 8)[None, :]
    Me = jnp.kron(eye8, W2e)
    b2e_t = jnp.full((1, 8), b2e[0], jnp.float32)

    eidx, vfir, vsec, lpe = _tc2(
        hs_s.reshape(E8, D), hs_d.reshape(E8, D), noise8,
        src.reshape(E8, 8), dst.reshape(E8, 8), gsum,
        W1e_top_t, b1e_t, Me, b2e_t)

    Wg_t = jnp.tile(W1t[:D], (1, 8))
    Ws_t = jnp.tile(W1t[D:2 * D], (1, 8))
    Wf_t = jnp.tile(W1t[2 * D:3 * D], (1, 8))
    b1t_t = jnp.tile(b1t, 8)[None, :]
    Mt = jnp.kron(eye8, W2t)
    b2t_t = jnp.full((1, 8), b2t[0], jnp.float32)

    tl8 = _tc3(nodeT.reshape(N // 8, D), emb, gsum,
               vfir.reshape(1), vsec.reshape(1),
               Wg_t, Ws_t, Wf_t, b1t_t, Mt, b2t_t)

    vf16 = jnp.broadcast_to(vfir.reshape(1), (16,)).astype(jnp.int32)
    resi, resf = _sc3(src, dst, vf16, tl8.reshape(N), gnoise)

    v_thi = resi[0]
    lp3 = (resf[0] - resf[1]) - jnp.log(resf[2])

    action = jnp.stack([vfir[0].astype(jnp.int32),
                        vsec[0].astype(jnp.int32),
                        v_thi.astype(jnp.int32)])
    return action, (lpe[0] + lp3).astype(jnp.float32)


# speedup vs baseline: 13.5420x; 1.0438x over previous
import functools

import jax
import jax.numpy as jnp
import numpy as np
from jax import lax
from jax.experimental import pallas as pl
from jax.experimental.pallas import tpu as pltpu
from jax.experimental.pallas import tpu_sc as plsc

N, E, D, H = 10000, 160000, 128, 16
NC, NS = 2, 16
NW = NC * NS
EPW = E // NW
EC1 = 152
EC1T = 136
NPT = N // NS
E8 = E // 8
ER = 2000
BN = 2000
NINF = np.float32(-np.inf)

_mesh = lambda: plsc.VectorSubcoreMesh(core_axis_name="c", subcore_axis_name="s")


def _sc1_body(src_hbm, dst_hbm, x_hbm, out_hbm, idx_s, idx_d, rows0, rows1,
              agg_sh, sg0, sg1, ss0, ss1):
    cid = lax.axis_index("c")
    sid = lax.axis_index("s")
    base = pl.multiple_of(cid * (E // NC) + sid * EPW, 8)

    pltpu.sync_copy(src_hbm.at[pl.ds(base, EPW)], idx_s)
    pltpu.sync_copy(dst_hbm.at[pl.ds(base, EPW)], idx_d)

    def gidx(c, size=EC1):
        return idx_s.at[pl.ds(c * EC1, size)]

    def didx(c, size=EC1):
        return idx_d.at[pl.ds(c * EC1, size)]

    def zrow(i, c):
        for j in range(D // 16):
            rows0[i, pl.ds(j * 16, 16)] = jnp.zeros((16,), jnp.float32)
        return c

    lax.fori_loop(0, EC1, zrow, 0)
    for off, size in ((0, 152), (152, 152), (304, 152), (456, 152), (608, 16)):
        pltpu.sync_copy(rows0.at[pl.ds(0, size)],
                        agg_sh.at[pl.ds(sid * 624 + off, size)])

    @pl.when(sid == NS - 1)
    def _():
        pltpu.sync_copy(rows0.at[pl.ds(0, 16)], agg_sh.at[pl.ds(624 * NS, 16)])

    plsc.subcore_barrier()

    pltpu.async_copy(x_hbm.at[gidx(0)], rows0, sg0)

    def body(j, c):
        c0 = 2 * j
        pltpu.make_async_copy(x_hbm.at[gidx(c0)], rows0, sg0).wait()
        s0 = pltpu.async_copy(rows0, agg_sh.at[didx(c0)], ss0, add=True)

        @pl.when(j > 0)
        def _():
            pltpu.make_async_copy(rows1, agg_sh.at[didx(c0 - 1)], ss1).wait()

        pltpu.async_copy(x_hbm.at[gidx(c0 + 1)], rows1, sg1)
        pltpu.make_async_copy(x_hbm.at[gidx(c0 + 1)], rows1, sg1).wait()
        pltpu.async_copy(rows1, agg_sh.at[didx(c0 + 1)], ss1, add=True)
        s0.wait()

        @pl.when(j < (EPW // EC1) // 2 - 1)
        def _():
            pltpu.async_copy(x_hbm.at[gidx(c0 + 2)], rows0, sg0)

        return c

    lax.fori_loop(0, (EPW // EC1) // 2, body, 0)
    pltpu.make_async_copy(rows1, agg_sh.at[didx(EPW // EC1 - 1)], ss1).wait()

    nc = EPW // EC1
    pltpu.async_copy(x_hbm.at[gidx(nc, EC1T)], rows0.at[pl.ds(0, EC1T)], sg0).wait()
    pltpu.async_copy(rows0.at[pl.ds(0, EC1T)], agg_sh.at[didx(nc, EC1T)],
                     ss0, add=True).wait()

    plsc.subcore_barrier()

    b = sid * 624
    for off, size in ((0, 152), (152, 152), (304, 152), (456, 152), (608, 16)):
        pltpu.sync_copy(agg_sh.at[pl.ds(b + off, size)], rows0.at[pl.ds(0, size)])
        pltpu.sync_copy(rows0.at[pl.ds(0, size)],
                        out_hbm.at[pl.ds(cid * N + b + off, size)])

    @pl.when(sid == NS - 1)
    def _():
        t = 624 * NS
        pltpu.sync_copy(agg_sh.at[pl.ds(t, 16)], rows0.at[pl.ds(0, 16)])
        pltpu.sync_copy(rows0.at[pl.ds(0, 16)], out_hbm.at[pl.ds(cid * N + t, 16)])


def _sc1(src, dst, x):
    k = pl.kernel(
        _sc1_body,
        out_type=jax.ShapeDtypeStruct((NC * N, D), jnp.float32),
        mesh=_mesh(),
        compiler_params=pltpu.CompilerParams(use_tc_tiling_on_sc=False),
        scratch_types=[
            pltpu.VMEM((EPW,), jnp.int32),
            pltpu.VMEM((EPW,), jnp.int32),
            pltpu.VMEM((EC1, D), jnp.float32),
            pltpu.VMEM((EC1, D), jnp.float32),
            pltpu.VMEM_SHARED((N, D), jnp.float32),
            pltpu.SemaphoreType.DMA,
            pltpu.SemaphoreType.DMA,
            pltpu.SemaphoreType.DMA,
            pltpu.SemaphoreType.DMA,
        ],
    )
    return k(src, dst, x)


def _tc1_body(x_ref, a0_ref, a1_ref, wg_ref, bg_ref, whe_ref, wht_ref, nz_ref,
              emb_ref, nh_ref, nt_ref, gs_ref, nzo_ref):
    nzo_ref[...] = nz_ref[...]
    xb = x_ref[...] + a0_ref[...] + a1_ref[...]
    emb = jnp.maximum(
        jnp.dot(xb, wg_ref[...], preferred_element_type=jnp.float32) + bg_ref[...],
        0.0)
    emb_ref[...] = emb
    nh_ref[...] = jnp.dot(emb, whe_ref[...], preferred_element_type=jnp.float32)
    nt_ref[...] = jnp.dot(emb, wht_ref[...], preferred_element_type=jnp.float32)
    s = jnp.sum(emb, axis=0, keepdims=True)

    @pl.when(pl.program_id(0) == 0)
    def _():
        gs_ref[...] = s

    @pl.when(pl.program_id(0) != 0)
    def _():
        gs_ref[...] = gs_ref[...] + s


def _tc1(x, agg2, W_gnn, b_gnn, W1e_bot, W1t_bot, noise8):
    nsteps = N // BN
    nzr = E8 // nsteps
    return pl.pallas_call(
        _tc1_body,
        grid=(nsteps,),
        in_specs=[
            pl.BlockSpec((BN, D), lambda i: (i, 0)),
            pl.BlockSpec((BN, D), lambda i: (i, 0)),
            pl.BlockSpec((BN, D), lambda i: (i + nsteps, 0)),
            pl.BlockSpec((D, D), lambda i: (0, 0)),
            pl.BlockSpec((1, D), lambda i: (0, 0)),
            pl.BlockSpec((D, H), lambda i: (0, 0)),
            pl.BlockSpec((D, H), lambda i: (0, 0)),
            pl.BlockSpec((nzr, 8), lambda i: (i, 0)),
        ],
        out_specs=[
            pl.BlockSpec((BN, D), lambda i: (i, 0)),
            pl.BlockSpec((BN, H), lambda i: (i, 0)),
            pl.BlockSpec((BN, H), lambda i: (i, 0)),
            pl.BlockSpec((1, D), lambda i: (0, 0)),
            pl.BlockSpec((nzr, 8), lambda i: (i, 0)),
        ],
        out_shape=[
            jax.ShapeDtypeStruct((N, D), jnp.float32),
            jax.ShapeDtypeStruct((N, H), jnp.float32),
            jax.ShapeDtypeStruct((N, H), jnp.float32),
            jax.ShapeDtypeStruct((1, D), jnp.float32),
            jax.ShapeDtypeStruct((E8, 8), jnp.float32),
        ],
    )(x, agg2, agg2, W_gnn, b_gnn, W1e_bot, W1t_bot, noise8)


def _sc2_body(src_hbm, dst_hbm, nh_hbm, outs_hbm, outd_hbm, idx, rows, sem):
    cid = lax.axis_index("c")
    sid = lax.axis_index("s")
    base = pl.multiple_of((cid * NS + sid) * EPW, 8)
    pltpu.sync_copy(src_hbm.at[pl.ds(base, EPW)], idx)
    pltpu.async_copy(nh_hbm.at[idx], rows, sem).wait()
    pltpu.sync_copy(rows, outs_hbm.at[pl.ds(base, EPW)])
    pltpu.sync_copy(dst_hbm.at[pl.ds(base, EPW)], idx)
    pltpu.async_copy(nh_hbm.at[idx], rows, sem).wait()
    pltpu.sync_copy(rows, outd_hbm.at[pl.ds(base, EPW)])


def _sc2(src, dst, nodeH):
    k = pl.kernel(
        _sc2_body,
        out_type=(jax.ShapeDtypeStruct((E, H), jnp.float32),
                  jax.ShapeDtypeStruct((E, H), jnp.float32)),
        mesh=_mesh(),
        compiler_params=pltpu.CompilerParams(use_tc_tiling_on_sc=False),
        scratch_types=[
            pltpu.VMEM((EPW,), jnp.int32),
            pltpu.VMEM((EPW, H), jnp.float32),
            pltpu.SemaphoreType.DMA,
        ],
    )
    return k(src, dst, nodeH)


def _tc2_body(hss, hsd, nz, sr, dr, gs, w1t, b1t, me, b2,
              eidx_o, vfir_o, vsec_o, lpe_o, Mr, Sr, Bv, Bl, Bi, Bs, Bd):
    step = pl.program_id(0)
    ER16 = ER * 8 // 128

    @pl.when(step == 0)
    def _():
        Mr[0] = NINF
        Sr[0] = jnp.float32(0.0)
        Bv[0] = NINF
        Bl[0] = jnp.float32(0.0)
        Bi[0] = jnp.int32(0)
        Bs[0] = jnp.int32(0)
        Bd[0] = jnp.int32(0)

    cE = jnp.dot(gs[...] / N, w1t[...], preferred_element_type=jnp.float32) + b1t[...]
    sg = jax.nn.sigmoid(hss[...] + hsd[...] + cE)
    lg = jnp.dot(sg, me[...], preferred_element_type=jnp.float32) + b2[...]
    y = lg + nz[...]

    bm = jnp.max(y)
    lmax = jnp.max(lg)
    newM = jnp.maximum(Mr[0], lmax)
    Sr[0] = Sr[0] * jnp.exp(Mr[0] - newM) + jnp.sum(jnp.exp(lg - newM))
    Mr[0] = newM

    @pl.when(bm > Bv[0])
    def _():
        rid = lax.broadcasted_iota(jnp.int32, (ER, 8), 0)
        cid = lax.broadcasted_iota(jnp.int32, (ER, 8), 1)
        ids = (step * ER + rid) * 8 + cid
        bi = jnp.min(jnp.where(y == bm, ids, jnp.int32(2**31 - 1)))
        selm = ids == bi
        Bv[0] = bm
        Bi[0] = bi
        Bl[0] = jnp.max(jnp.where(selm, lg, NINF))
        Bs[0] = jnp.max(jnp.where(selm, sr[...], jnp.int32(-1)))
        Bd[0] = jnp.max(jnp.where(selm, dr[...], jnp.int32(-1)))

    eidx_o[0] = Bi[0]
    vfir_o[0] = Bd[0]
    vsec_o[0] = Bs[0]
    lpe_o[0] = Bl[0] - Mr[0] - jnp.log(Sr[0])


def _tc2(hs_s, hs_d, noise8, src8, dst8, gsum, W1e_top_t, b1e_t, Me, b2e_t):
    ER16 = ER * 8 // 128
    return pl.pallas_call(
        _tc2_body,
        grid=(E8 // ER,),
        in_specs=[
            pl.BlockSpec((ER, D), lambda i: (i, 0)),
            pl.BlockSpec((ER, D), lambda i: (i, 0)),
            pl.BlockSpec((ER, 8), lambda i: (i, 0)),
            pl.BlockSpec((ER, 8), lambda i: (i, 0)),
            pl.BlockSpec((ER, 8), lambda i: (i, 0)),
            pl.BlockSpec((1, D), lambda i: (0, 0)),
            pl.BlockSpec((D, D), lambda i: (0, 0)),
            pl.BlockSpec((1, D), lambda i: (0, 0)),
            pl.BlockSpec((D, 8), lambda i: (0, 0)),
            pl.BlockSpec((1, 8), lambda i: (0, 0)),
        ],
        out_specs=[
            pl.BlockSpec(memory_space=pltpu.SMEM),
            pl.BlockSpec(memory_space=pltpu.SMEM),
            pl.BlockSpec(memory_space=pltpu.SMEM),
            pl.BlockSpec(memory_space=pltpu.SMEM),
        ],
        out_shape=[
            jax.ShapeDtypeStruct((1,), jnp.int32),
            jax.ShapeDtypeStruct((1,), jnp.int32),
            jax.ShapeDtypeStruct((1,), jnp.int32),
            jax.ShapeDtypeStruct((1,), jnp.float32),
        ],
        scratch_shapes=[
            pltpu.SMEM((1,), jnp.float32),
            pltpu.SMEM((1,), jnp.float32),
            pltpu.SMEM((1,), jnp.float32),
            pltpu.SMEM((1,), jnp.float32),
            pltpu.SMEM((1,), jnp.int32),
            pltpu.SMEM((1,), jnp.int32),
            pltpu.SMEM((1,), jnp.int32),
        ],
    )(hs_s, hs_d, noise8, src8, dst8, gsum, W1e_top_t, b1e_t, Me, b2e_t)


def _tc3_body(ntr, emb, gs, vf_s, vs_s, wgt, wst, wft, b1t_t, mt, b2t_t, out):
    vf = vf_s[0]
    vs = vs_s[0]
    ef = emb[pl.ds(vf, 1), :]
    es = emb[pl.ds(vs, 1), :]
    c3 = (jnp.dot(gs[...] / N, wgt[...], preferred_element_type=jnp.float32)
          + jnp.dot(ef + es, wst[...], preferred_element_type=jnp.float32)
          + jnp.dot(ef, wft[...], preferred_element_type=jnp.float32)
          + b1t_t[...])
    sg = jax.nn.sigmoid(ntr[...] + c3)
    out[...] = jnp.dot(sg, mt[...], preferred_element_type=jnp.float32) + b2t_t[...]


def _tc3(ntr, emb, gsum, vfir, vsec, Wg_t, Ws_t, Wf_t, b1t_t, Mt, b2t_t):
    return pl.pallas_call(
        _tc3_body,
        in_specs=[
            pl.BlockSpec((N // 8, D), lambda: (0, 0)),
            pl.BlockSpec((N, D), lambda: (0, 0)),
            pl.BlockSpec((1, D), lambda: (0, 0)),
            pl.BlockSpec(memory_space=pltpu.SMEM),
            pl.BlockSpec(memory_space=pltpu.SMEM),
            pl.BlockSpec((D, D), lambda: (0, 0)),
            pl.BlockSpec((D, D), lambda: (0, 0)),
            pl.BlockSpec((D, D), lambda: (0, 0)),
            pl.BlockSpec((1, D), lambda: (0, 0)),
            pl.BlockSpec((D, 8), lambda: (0, 0)),
            pl.BlockSpec((1, 8), lambda: (0, 0)),
        ],
        out_specs=pl.BlockSpec((N // 8, 8), lambda: (0, 0)),
        out_shape=jax.ShapeDtypeStruct((N // 8, 8), jnp.float32),
    )(ntr, emb, gsum, vfir, vsec, Wg_t, Ws_t, Wf_t, b1t_t, Mt, b2t_t)


def _sc3_body(src_hbm, dst_hbm, vf_hbm, lg_hbm, gn_hbm, resi_hbm, resf_hbm,
              dstb, srcb, valb, cnt_v, lg_v, gn_v, vf_v, v16, i16b,
              resv_i, resv_f, fpub, c256_v, f256_v,
              cnt_sh, cnts_sh, fres_sh, ires_sh, sem):
    cid = lax.axis_index("c")
    sid = lax.axis_index("s")
    on0 = cid == 0
    lanes = lax.iota(jnp.int32, 16)

    @pl.when(jnp.logical_and(on0, sid == 0))
    def _():
        def z(i, c):
            valb[pl.ds(i * 16, 16)] = jnp.zeros((16,), jnp.int32)
            return c
        lax.fori_loop(0, N // 16, z, 0)
        pltpu.sync_copy(valb.at[pl.ds(0, N)], cnt_sh)

    plsc.subcore_barrier()

    @pl.when(on0)
    def _():
        pltpu.sync_copy(vf_hbm, vf_v)
        vf = vf_v[...]
        eb = pl.multiple_of(sid * (E // NS), 8)
        pltpu.sync_copy(dst_hbm.at[pl.ds(eb, E // NS)], dstb)
        pltpu.sync_copy(src_hbm.at[pl.ds(eb, E // NS)], srcb)

        def cmp(i, c):
            d = dstb[pl.ds(i * 16, 16)]
            valb[pl.ds(i * 16, 16)] = jnp.where(d == vf, 1, 0).astype(jnp.int32)
            return c

        lax.fori_loop(0, (E // NS) // 16, cmp, 0)
        pltpu.sync_copy(valb, cnt_sh.at[srcb], add=True)

        @pl.when(sid == 0)
        def _():
            v16[...] = jnp.where(lanes == 0, 1, 0).astype(jnp.int32)
            i16b[...] = vf
            pltpu.sync_copy(v16, cnt_sh.at[i16b], add=True)

    plsc.subcore_barrier()

    IMAX = jnp.int32(2**31 - 1)

    @pl.when(on0)
    def _():
        base = sid * 624
        nit = jnp.where(sid == NS - 1, 40, 39)
        pltpu.sync_copy(cnt_sh.at[pl.ds(base, 640)], cnt_v)
        pltpu.sync_copy(lg_hbm.at[pl.ds(base, 640)], lg_v)

        def pc(i, acc):
            c = cnt_v[pl.ds(i * 16, 16)]
            return acc + plsc.all_reduce_population_count(c == 0)

        cnt_loc = lax.fori_loop(0, nit, pc, jnp.zeros((16,), jnp.int32))
        v16[...] = cnt_loc
        pltpu.sync_copy(v16, cnts_sh.at[sid])

    plsc.subcore_barrier()

    @pl.when(on0)
    def _():
        base = sid * 624
        nit = jnp.where(sid == NS - 1, 40, 39)
        pltpu.sync_copy(cnts_sh, c256_v)
        c16 = plsc.load_gather(c256_v, [lanes, lanes])
        pref = plsc.cumsum(c16) - c16
        my_pref = jnp.sum(jnp.where(lanes == sid, pref, 0), axis=0)
        al = (my_pref // 8) * 8
        off = jnp.broadcast_to(my_pref - al, (16,))
        pltpu.sync_copy(gn_hbm.at[pl.ds(pl.multiple_of(al, 8), 648)], gn_v)

        def pA(i, m):
            l = lg_v[pl.ds(i * 16, 16)]
            c = cnt_v[pl.ds(i * 16, 16)]
            return jnp.maximum(m, jnp.where(c > 0, NINF, l))

        mreg = lax.fori_loop(0, nit, pA, jnp.full((16,), NINF, jnp.float32))
        mx_loc = jnp.max(mreg, axis=0)
        mxv = jnp.broadcast_to(mx_loc, (16,))

        def pB(i, carry):
            cp, se, bv, bn, bl = carry
            l = lg_v[pl.ds(i * 16, 16)]
            c = cnt_v[pl.ds(i * 16, 16)]
            validb = c == 0
            vi = jnp.where(validb, 1, 0).astype(jnp.int32)
            incl = plsc.cumsum(vi)
            pos = cp + incl - vi
            gn = plsc.load_gather(gn_v, [pos])
            val = jnp.where(validb, l + gn, NINF)
            upd = val > bv
            bv = jnp.where(upd, val, bv)
            bn = jnp.where(upd, lanes + base + i * 16, bn)
            bl = jnp.where(upd, l, bl)
            se = se + jnp.where(validb, jnp.exp(l - mxv), jnp.float32(0.0))
            cp = cp + plsc.all_reduce_population_count(validb)
            return (cp, se, bv, bn, bl)

        z16i = jnp.zeros((16,), jnp.int32)
        cp, se, bv, bn, bl = lax.fori_loop(
            0, nit, pB,
            (off, jnp.zeros((16,), jnp.float32), jnp.full((16,), NINF, jnp.float32),
             z16i, jnp.zeros((16,), jnp.float32)))

        M_loc = jnp.max(bv, axis=0)
        eqv = bv == jnp.broadcast_to(M_loc, (16,))
        wn_loc = jnp.min(jnp.where(eqv, bn, IMAX), axis=0)
        wnv_loc = jnp.broadcast_to(wn_loc, (16,))
        bl_loc = jnp.max(jnp.where(jnp.logical_and(eqv, bn == wnv_loc), bl, NINF),
                         axis=0)
        S_loc = jnp.sum(se, axis=0)
        fpub[...] = jnp.where(lanes == 0, mxv,
                     jnp.where(lanes == 1, jnp.broadcast_to(S_loc, (16,)),
                      jnp.where(lanes == 2, jnp.broadcast_to(M_loc, (16,)),
                                jnp.broadcast_to(bl_loc, (16,)))))
        pltpu.sync_copy(fpub, fres_sh.at[sid])
        i16b[...] = wnv_loc
        pltpu.sync_copy(i16b, ires_sh.at[sid])

    plsc.subcore_barrier()

    @pl.when(jnp.logical_and(on0, sid == 0))
    def _():
        pltpu.sync_copy(fres_sh, f256_v)
        pltpu.sync_copy(ires_sh, c256_v)
        z16 = jnp.zeros((16,), jnp.int32)
        mx_t = plsc.load_gather(f256_v, [lanes, z16])
        S_t = plsc.load_gather(f256_v, [lanes, z16 + 1])
        M_t = plsc.load_gather(f256_v, [lanes, z16 + 2])
        bl_t = plsc.load_gather(f256_v, [lanes, z16 + 3])
        wn_t = plsc.load_gather(c256_v, [lanes, z16])
        mx = jnp.max(mx_t, axis=0)
        mxv = jnp.broadcast_to(mx, (16,))
        S = jnp.sum(S_t * jnp.exp(mx_t - mxv), axis=0)
        M = jnp.max(M_t, axis=0)
        eqt = M_t == jnp.broadcast_to(M, (16,))
        wn = jnp.min(jnp.where(eqt, wn_t, IMAX), axis=0)
        wnv = jnp.broadcast_to(wn, (16,))
        blw = jnp.max(jnp.where(jnp.logical_and(eqt, wn_t == wnv), bl_t, NINF),
                      axis=0)
        resv_i[...] = wnv
        resv_f[...] = jnp.where(lanes == 0, jnp.broadcast_to(blw, (16,)),
                                jnp.where(lanes == 1, mxv,
                                          jnp.broadcast_to(S, (16,))))
        pltpu.sync_copy(resv_i, resi_hbm)
        pltpu.sync_copy(resv_f, resf_hbm)


def _sc3(src, dst, vf16, lg, gn):
    k = pl.kernel(
        _sc3_body,
        out_type=(jax.ShapeDtypeStruct((16,), jnp.int32),
                  jax.ShapeDtypeStruct((16,), jnp.float32)),
        mesh=_mesh(),
        compiler_params=pltpu.CompilerParams(needs_layout_passes=False,
                                             use_tc_tiling_on_sc=False),
        scratch_types=[
            pltpu.VMEM((E // NS,), jnp.int32),
            pltpu.VMEM((E // NS,), jnp.int32),
            pltpu.VMEM((E // NS,), jnp.int32),
            pltpu.VMEM((640,), jnp.int32),
            pltpu.VMEM((640,), jnp.float32),
            pltpu.VMEM((648,), jnp.float32),
            pltpu.VMEM((16,), jnp.int32),
            pltpu.VMEM((16,), jnp.int32),
            pltpu.VMEM((16,), jnp.int32),
            pltpu.VMEM((16,), jnp.int32),
            pltpu.VMEM((16,), jnp.float32),
            pltpu.VMEM((16,), jnp.float32),
            pltpu.VMEM((16, 16), jnp.int32),
            pltpu.VMEM((16, 16), jnp.float32),
            pltpu.VMEM_SHARED((N,), jnp.int32),
            pltpu.VMEM_SHARED((16, 16), jnp.int32),
            pltpu.VMEM_SHARED((16, 16), jnp.float32),
            pltpu.VMEM_SHARED((16, 16), jnp.int32),
            pltpu.SemaphoreType.DMA,
        ],
    )
    return k(src, dst, vf16, lg, gn)


def kernel(x, edge_index, W_gnn, b_gnn, W1e, b1e, W2e, b2e, W1t, b1t, W2t, b2t):
    src = edge_index[0]
    dst = edge_index[1]

    noiseE = jax.random.gumbel(jax.random.key(42), (E,), jnp.float32)
    gnoise = jax.random.gumbel(jax.random.key(43), (N,), jnp.float32)

    agg2 = _sc1(src, dst, x)

    emb, nodeH, nodeT, gsum, noise8 = _tc1(x, agg2, W_gnn, b_gnn[None, :],
                                           W1e[D:], W1t[3 * D:],
                                           noiseE.reshape(E8, 8))

    hs_s, hs_d = _sc2(src, dst, nodeH)

    eye8 = jnp.eye(8, dtype=jnp.float32)
    W1e_top_t = jnp.tile(W1e[:D], (1, 8))
    b1e_t = jnp.tile(b1e, 8)[None, :]
    Me = jnp.kron(eye8, W2e)
    b2e_t = jnp.full((1, 8), b2e[0], jnp.float32)

    eidx, vfir, vsec, lpe = _tc2(
        hs_s.reshape(E8, D), hs_d.reshape(E8, D), noise8,
        src.reshape(E8, 8), dst.reshape(E8, 8), gsum,
        W1e_top_t, b1e_t, Me, b2e_t)

    Wg_t = jnp.tile(W1t[:D], (1, 8))
    Ws_t = jnp.tile(W1t[D:2 * D], (1, 8))
    Wf_t = jnp.tile(W1t[2 * D:3 * D], (1, 8))
    b1t_t = jnp.tile(b1t, 8)[None, :]
    Mt = jnp.kron(eye8, W2t)
    b2t_t = jnp.full((1, 8), b2t[0], jnp.float32)

    tl8 = _tc3(nodeT.reshape(N // 8, D), emb, gsum,
               vfir.reshape(1), vsec.reshape(1),
               Wg_t, Ws_t, Wf_t, b1t_t, Mt, b2t_t)

    vf16 = jnp.broadcast_to(vfir.reshape(1), (16,)).astype(jnp.int32)
    gn_pad = jnp.concatenate([gnoise, jnp.zeros((656,), jnp.float32)])
    resi, resf = _sc3(src, dst, vf16, tl8.reshape(N), gn_pad)

    v_thi = resi[0]
    lp3 = (resf[0] - resf[1]) - jnp.log(resf[2])

    action = jnp.stack([vfir[0].astype(jnp.int32),
                        vsec[0].astype(jnp.int32),
                        v_thi.astype(jnp.int32)])
    return action, (lpe[0] + lp3).astype(jnp.float32)


# speedup vs baseline: 13.8048x; 1.0194x over previous
import functools

import jax
import jax.numpy as jnp
import numpy as np
from jax import lax
from jax.experimental import pallas as pl
from jax.experimental.pallas import tpu as pltpu
from jax.experimental.pallas import tpu_sc as plsc

N, E, D, H = 10000, 160000, 128, 16
NC, NS = 2, 16
NW = NC * NS
EPW = E // NW
EC1 = 152
EC1T = 136
NPT = N // NS
E8 = E // 8
ER = 2000
BN = 2000
NINF = np.float32(-np.inf)

_mesh = lambda: plsc.VectorSubcoreMesh(core_axis_name="c", subcore_axis_name="s")


def _sc1_body(edge_hbm, x_hbm, out_hbm, idx_s, idx_d, rows0, rows1,
              agg_sh, sg0, sg1, ss0, ss1):
    cid = lax.axis_index("c")
    sid = lax.axis_index("s")
    base = pl.multiple_of(cid * (E // NC) + sid * EPW, 8)

    pltpu.sync_copy(edge_hbm.at[0, pl.ds(base, EPW)], idx_s)
    pltpu.sync_copy(edge_hbm.at[1, pl.ds(base, EPW)], idx_d)

    def gidx(c, size=EC1):
        return idx_s.at[pl.ds(c * EC1, size)]

    def didx(c, size=EC1):
        return idx_d.at[pl.ds(c * EC1, size)]

    def zrow(i, c):
        for j in range(D // 16):
            rows0[i, pl.ds(j * 16, 16)] = jnp.zeros((16,), jnp.float32)
        return c

    lax.fori_loop(0, EC1, zrow, 0)
    for off, size in ((0, 152), (152, 152), (304, 152), (456, 152), (608, 16)):
        pltpu.sync_copy(rows0.at[pl.ds(0, size)],
                        agg_sh.at[pl.ds(sid * 624 + off, size)])

    @pl.when(sid == NS - 1)
    def _():
        pltpu.sync_copy(rows0.at[pl.ds(0, 16)], agg_sh.at[pl.ds(624 * NS, 16)])

    plsc.subcore_barrier()

    pltpu.async_copy(x_hbm.at[gidx(0)], rows0, sg0)

    def body(j, c):
        c0 = 2 * j
        pltpu.make_async_copy(x_hbm.at[gidx(c0)], rows0, sg0).wait()
        s0 = pltpu.async_copy(rows0, agg_sh.at[didx(c0)], ss0, add=True)

        @pl.when(j > 0)
        def _():
            pltpu.make_async_copy(rows1, agg_sh.at[didx(c0 - 1)], ss1).wait()

        pltpu.async_copy(x_hbm.at[gidx(c0 + 1)], rows1, sg1)
        pltpu.make_async_copy(x_hbm.at[gidx(c0 + 1)], rows1, sg1).wait()
        pltpu.async_copy(rows1, agg_sh.at[didx(c0 + 1)], ss1, add=True)
        s0.wait()

        @pl.when(j < (EPW // EC1) // 2 - 1)
        def _():
            pltpu.async_copy(x_hbm.at[gidx(c0 + 2)], rows0, sg0)

        return c

    lax.fori_loop(0, (EPW // EC1) // 2, body, 0)
    pltpu.make_async_copy(rows1, agg_sh.at[didx(EPW // EC1 - 1)], ss1).wait()

    nc = EPW // EC1
    pltpu.async_copy(x_hbm.at[gidx(nc, EC1T)], rows0.at[pl.ds(0, EC1T)], sg0).wait()
    pltpu.async_copy(rows0.at[pl.ds(0, EC1T)], agg_sh.at[didx(nc, EC1T)],
                     ss0, add=True).wait()

    plsc.subcore_barrier()

    b = sid * 624
    for off, size in ((0, 152), (152, 152), (304, 152), (456, 152), (608, 16)):
        pltpu.sync_copy(agg_sh.at[pl.ds(b + off, size)], rows0.at[pl.ds(0, size)])
        pltpu.sync_copy(rows0.at[pl.ds(0, size)],
                        out_hbm.at[pl.ds(cid * N + b + off, size)])

    @pl.when(sid == NS - 1)
    def _():
        t = 624 * NS
        pltpu.sync_copy(agg_sh.at[pl.ds(t, 16)], rows0.at[pl.ds(0, 16)])
        pltpu.sync_copy(rows0.at[pl.ds(0, 16)], out_hbm.at[pl.ds(cid * N + t, 16)])


def _sc1(edge_index, x):
    k = pl.kernel(
        _sc1_body,
        out_type=jax.ShapeDtypeStruct((NC * N, D), jnp.float32),
        mesh=_mesh(),
        compiler_params=pltpu.CompilerParams(use_tc_tiling_on_sc=False),
        scratch_types=[
            pltpu.VMEM((EPW,), jnp.int32),
            pltpu.VMEM((EPW,), jnp.int32),
            pltpu.VMEM((EC1, D), jnp.float32),
            pltpu.VMEM((EC1, D), jnp.float32),
            pltpu.VMEM_SHARED((N, D), jnp.float32),
            pltpu.SemaphoreType.DMA,
            pltpu.SemaphoreType.DMA,
            pltpu.SemaphoreType.DMA,
            pltpu.SemaphoreType.DMA,
        ],
    )
    return k(edge_index, x)


def _tc1_body(x_ref, a0_ref, a1_ref, wg_ref, bg_ref, whe_ref, wht_ref, nz_ref,
              emb_ref, nh_ref, nt_ref, gs_ref, nzo_ref):
    nzo_ref[...] = nz_ref[...]
    xb = x_ref[...] + a0_ref[...] + a1_ref[...]
    emb = jnp.maximum(
        jnp.dot(xb, wg_ref[...], preferred_element_type=jnp.float32) + bg_ref[...],
        0.0)
    emb_ref[...] = emb
    nh_ref[...] = jnp.dot(emb, whe_ref[...], preferred_element_type=jnp.float32)
    nt_ref[...] = jnp.dot(emb, wht_ref[...], preferred_element_type=jnp.float32)
    s = jnp.sum(emb, axis=0, keepdims=True)

    @pl.when(pl.program_id(0) == 0)
    def _():
        gs_ref[...] = s

    @pl.when(pl.program_id(0) != 0)
    def _():
        gs_ref[...] = gs_ref[...] + s


def _tc1(x, agg2, W_gnn, b_gnn, W1e_bot, W1t_bot, noise8):
    nsteps = N // BN
    nzr = E8 // nsteps
    return pl.pallas_call(
        _tc1_body,
        grid=(nsteps,),
        in_specs=[
            pl.BlockSpec((BN, D), lambda i: (i, 0)),
            pl.BlockSpec((BN, D), lambda i: (i, 0)),
            pl.BlockSpec((BN, D), lambda i: (i + nsteps, 0)),
            pl.BlockSpec((D, D), lambda i: (0, 0)),
            pl.BlockSpec((1, D), lambda i: (0, 0)),
            pl.BlockSpec((D, H), lambda i: (0, 0)),
            pl.BlockSpec((D, H), lambda i: (0, 0)),
            pl.BlockSpec((nzr, 8), lambda i: (i, 0)),
        ],
        out_specs=[
            pl.BlockSpec((BN, D), lambda i: (i, 0)),
            pl.BlockSpec((BN, H), lambda i: (i, 0)),
            pl.BlockSpec((BN, H), lambda i: (i, 0)),
            pl.BlockSpec((1, D), lambda i: (0, 0)),
            pl.BlockSpec((nzr, 8), lambda i: (i, 0)),
        ],
        out_shape=[
            jax.ShapeDtypeStruct((N, D), jnp.float32),
            jax.ShapeDtypeStruct((N, H), jnp.float32),
            jax.ShapeDtypeStruct((N, H), jnp.float32),
            jax.ShapeDtypeStruct((1, D), jnp.float32),
            jax.ShapeDtypeStruct((E8, 8), jnp.float32),
        ],
    )(x, agg2, agg2, W_gnn, b_gnn, W1e_bot, W1t_bot, noise8)


def _sc2_body(src_hbm, dst_hbm, nh_hbm, outs_hbm, outd_hbm,
              idx, rows_a, rows_b, sga, sgb, swa, swb):
    cid = lax.axis_index("c")
    sid = lax.axis_index("s")
    EA, EB = 2504, 2496
    base = pl.multiple_of((cid * NS + sid) * EPW, 8)

    pltpu.sync_copy(src_hbm.at[pl.ds(base, EPW)], idx)
    ga = pltpu.async_copy(nh_hbm.at[idx.at[pl.ds(0, EA)]], rows_a, sga)
    gb = pltpu.async_copy(nh_hbm.at[idx.at[pl.ds(EA, EB)]],
                          rows_b.at[pl.ds(0, EB)], sgb)
    ga.wait()
    wa = pltpu.async_copy(rows_a, outs_hbm.at[pl.ds(base, EA)], swa)
    gb.wait()
    wb = pltpu.async_copy(rows_b.at[pl.ds(0, EB)],
                          outs_hbm.at[pl.ds(base + EA, EB)], swb)
    pltpu.sync_copy(dst_hbm.at[pl.ds(base, EPW)], idx)
    wa.wait()
    ga = pltpu.async_copy(nh_hbm.at[idx.at[pl.ds(0, EA)]], rows_a, sga)
    wb.wait()
    gb = pltpu.async_copy(nh_hbm.at[idx.at[pl.ds(EA, EB)]],
                          rows_b.at[pl.ds(0, EB)], sgb)
    ga.wait()
    wa = pltpu.async_copy(rows_a, outd_hbm.at[pl.ds(base, EA)], swa)
    gb.wait()
    wb = pltpu.async_copy(rows_b.at[pl.ds(0, EB)],
                          outd_hbm.at[pl.ds(base + EA, EB)], swb)
    wa.wait()
    wb.wait()


def _sc2(src, dst, nodeH):
    k = pl.kernel(
        _sc2_body,
        out_type=(jax.ShapeDtypeStruct((E, H), jnp.float32),
                  jax.ShapeDtypeStruct((E, H), jnp.float32)),
        mesh=_mesh(),
        compiler_params=pltpu.CompilerParams(use_tc_tiling_on_sc=False),
        scratch_types=[
            pltpu.VMEM((EPW,), jnp.int32),
            pltpu.VMEM((2504, H), jnp.float32),
            pltpu.VMEM((2504, H), jnp.float32),
            pltpu.SemaphoreType.DMA,
            pltpu.SemaphoreType.DMA,
            pltpu.SemaphoreType.DMA,
            pltpu.SemaphoreType.DMA,
        ],
    )
    return k(src, dst, nodeH)


def _tc2_body(hss, hsd, nz, sr, dr, gs, w1t, b1t, me, b2,
              eidx_o, vfir_o, vsec_o, lpe_o, Mr, Sr, Bv, Bl, Bi, Bs, Bd):
    step = pl.program_id(0)
    ER16 = ER * 8 // 128

    @pl.when(step == 0)
    def _():
        Mr[0] = NINF
        Sr[0] = jnp.float32(0.0)
        Bv[0] = NINF
        Bl[0] = jnp.float32(0.0)
        Bi[0] = jnp.int32(0)
        Bs[0] = jnp.int32(0)
        Bd[0] = jnp.int32(0)

    cE = jnp.dot(gs[...] / N, w1t[...], preferred_element_type=jnp.float32) + b1t[...]
    sg = jax.nn.sigmoid(hss[...] + hsd[...] + cE)
    lg = jnp.dot(sg, me[...], preferred_element_type=jnp.float32) + b2[...]
    y = lg + nz[...]

    bm = jnp.max(y)
    lmax = jnp.max(lg)
    newM = jnp.maximum(Mr[0], lmax)
    Sr[0] = Sr[0] * jnp.exp(Mr[0] - newM) + jnp.sum(jnp.exp(lg - newM))
    Mr[0] = newM

    @pl.when(bm > Bv[0])
    def _():
        rid = lax.broadcasted_iota(jnp.int32, (ER, 8), 0)
        cid = lax.broadcasted_iota(jnp.int32, (ER, 8), 1)
        ids = (step * ER + rid) * 8 + cid
        bi = jnp.min(jnp.where(y == bm, ids, jnp.int32(2**31 - 1)))
        selm = ids == bi
        Bv[0] = bm
        Bi[0] = bi
        Bl[0] = jnp.max(jnp.where(selm, lg, NINF))
        Bs[0] = jnp.max(jnp.where(selm, sr[...], jnp.int32(-1)))
        Bd[0] = jnp.max(jnp.where(selm, dr[...], jnp.int32(-1)))

    eidx_o[0] = Bi[0]
    vfir_o[0] = Bd[0]
    vsec_o[0] = Bs[0]
    lpe_o[0] = Bl[0] - Mr[0] - jnp.log(Sr[0])


def _tc2(hs_s, hs_d, noise8, src8, dst8, gsum, W1e_top_t, b1e_t, Me, b2e_t):
    ER16 = ER * 8 // 128
    return pl.pallas_call(
        _tc2_body,
        grid=(E8 // ER,),
        in_specs=[
            pl.BlockSpec((ER, D), lambda i: (i, 0)),
            pl.BlockSpec((ER, D), lambda i: (i, 0)),
            pl.BlockSpec((ER, 8), lambda i: (i, 0)),
            pl.BlockSpec((ER, 8), lambda i: (i, 0)),
            pl.BlockSpec((ER, 8), lambda i: (i, 0)),
            pl.BlockSpec((1, D), lambda i: (0, 0)),
            pl.BlockSpec((D, D), lambda i: (0, 0)),
            pl.BlockSpec((1, D), lambda i: (0, 0)),
            pl.BlockSpec((D, 8), lambda i: (0, 0)),
            pl.BlockSpec((1, 8), lambda i: (0, 0)),
        ],
        out_specs=[
            pl.BlockSpec(memory_space=pltpu.SMEM),
            pl.BlockSpec(memory_space=pltpu.SMEM),
            pl.BlockSpec(memory_space=pltpu.SMEM),
            pl.BlockSpec(memory_space=pltpu.SMEM),
        ],
        out_shape=[
            jax.ShapeDtypeStruct((1,), jnp.int32),
            jax.ShapeDtypeStruct((1,), jnp.int32),
            jax.ShapeDtypeStruct((1,), jnp.int32),
            jax.ShapeDtypeStruct((1,), jnp.float32),
        ],
        scratch_shapes=[
            pltpu.SMEM((1,), jnp.float32),
            pltpu.SMEM((1,), jnp.float32),
            pltpu.SMEM((1,), jnp.float32),
            pltpu.SMEM((1,), jnp.float32),
            pltpu.SMEM((1,), jnp.int32),
            pltpu.SMEM((1,), jnp.int32),
            pltpu.SMEM((1,), jnp.int32),
        ],
    )(hs_s, hs_d, noise8, src8, dst8, gsum, W1e_top_t, b1e_t, Me, b2e_t)


def _tc3_body(ntr, emb, gs, vf_s, vs_s, wgt, wst, wft, b1t_t, mt, b2t_t, out):
    vf = vf_s[0]
    vs = vs_s[0]
    ef = emb[pl.ds(vf, 1), :]
    es = emb[pl.ds(vs, 1), :]
    c3 = (jnp.dot(gs[...] / N, wgt[...], preferred_element_type=jnp.float32)
          + jnp.dot(ef + es, wst[...], preferred_element_type=jnp.float32)
          + jnp.dot(ef, wft[...], preferred_element_type=jnp.float32)
          + b1t_t[...])
    sg = jax.nn.sigmoid(ntr[...] + c3)
    out[...] = jnp.dot(sg, mt[...], preferred_element_type=jnp.float32) + b2t_t[...]


def _tc3(ntr, emb, gsum, vfir, vsec, Wg_t, Ws_t, Wf_t, b1t_t, Mt, b2t_t):
    return pl.pallas_call(
        _tc3_body,
        in_specs=[
            pl.BlockSpec((N // 8, D), lambda: (0, 0)),
            pl.BlockSpec((N, D), lambda: (0, 0)),
            pl.BlockSpec((1, D), lambda: (0, 0)),
            pl.BlockSpec(memory_space=pltpu.SMEM),
            pl.BlockSpec(memory_space=pltpu.SMEM),
            pl.BlockSpec((D, D), lambda: (0, 0)),
            pl.BlockSpec((D, D), lambda: (0, 0)),
            pl.BlockSpec((D, D), lambda: (0, 0)),
            pl.BlockSpec((1, D), lambda: (0, 0)),
            pl.BlockSpec((D, 8), lambda: (0, 0)),
            pl.BlockSpec((1, 8), lambda: (0, 0)),
        ],
        out_specs=pl.BlockSpec((N // 8, 8), lambda: (0, 0)),
        out_shape=jax.ShapeDtypeStruct((N // 8, 8), jnp.float32),
    )(ntr, emb, gsum, vfir, vsec, Wg_t, Ws_t, Wf_t, b1t_t, Mt, b2t_t)


def _sc3_body(src_hbm, dst_hbm, vf_hbm, lg_hbm, gn_hbm, resi_hbm, resf_hbm,
              dstb, srcb, valb, cnt_v, lg_v, gn_v, vf_v, v16, i16b,
              resv_i, resv_f, fpub, c256_v, f256_v,
              cnt_sh, cnts_sh, fres_sh, ires_sh, sem):
    cid = lax.axis_index("c")
    sid = lax.axis_index("s")
    on0 = cid == 0
    lanes = lax.iota(jnp.int32, 16)

    @pl.when(jnp.logical_and(on0, sid == 0))
    def _():
        def z(i, c):
            valb[pl.ds(i * 16, 16)] = jnp.zeros((16,), jnp.int32)
            return c
        lax.fori_loop(0, N // 16, z, 0)
        pltpu.sync_copy(valb.at[pl.ds(0, N)], cnt_sh)

    plsc.subcore_barrier()

    @pl.when(on0)
    def _():
        pltpu.sync_copy(vf_hbm, vf_v)
        vf = vf_v[...]
        eb = pl.multiple_of(sid * (E // NS), 8)
        pltpu.sync_copy(dst_hbm.at[pl.ds(eb, E // NS)], dstb)
        pltpu.sync_copy(src_hbm.at[pl.ds(eb, E // NS)], srcb)

        def cmp(i, c):
            d = dstb[pl.ds(i * 16, 16)]
            valb[pl.ds(i * 16, 16)] = jnp.where(d == vf, 1, 0).astype(jnp.int32)
            return c

        lax.fori_loop(0, (E // NS) // 16, cmp, 0)
        pltpu.sync_copy(valb, cnt_sh.at[srcb], add=True)

        @pl.when(sid == 0)
        def _():
            v16[...] = jnp.where(lanes == 0, 1, 0).astype(jnp.int32)
            i16b[...] = vf
            pltpu.sync_copy(v16, cnt_sh.at[i16b], add=True)

    plsc.subcore_barrier()

    IMAX = jnp.int32(2**31 - 1)

    @pl.when(on0)
    def _():
        base = sid * 624
        nit = jnp.where(sid == NS - 1, 40, 39)
        pltpu.sync_copy(cnt_sh.at[pl.ds(base, 640)], cnt_v)
        pltpu.sync_copy(lg_hbm.at[pl.ds(base, 640)], lg_v)

        def pc(i, acc):
            c = cnt_v[pl.ds(i * 16, 16)]
            return acc + plsc.all_reduce_population_count(c == 0)

        cnt_loc = lax.fori_loop(0, nit, pc, jnp.zeros((16,), jnp.int32))
        v16[...] = cnt_loc
        pltpu.sync_copy(v16, cnts_sh.at[sid])

    plsc.subcore_barrier()

    @pl.when(on0)
    def _():
        base = sid * 624
        nit = jnp.where(sid == NS - 1, 40, 39)
        pltpu.sync_copy(cnts_sh, c256_v)
        c16 = plsc.load_gather(c256_v, [lanes, lanes])
        pref = plsc.cumsum(c16) - c16
        my_pref = jnp.sum(jnp.where(lanes == sid, pref, 0), axis=0)
        al = (my_pref // 8) * 8
        off = jnp.broadcast_to(my_pref - al, (16,))
        pltpu.sync_copy(gn_hbm.at[pl.ds(pl.multiple_of(al, 8), 648)], gn_v)

        def pA(i, m):
            l = lg_v[pl.ds(i * 16, 16)]
            c = cnt_v[pl.ds(i * 16, 16)]
            return jnp.maximum(m, jnp.where(c > 0, NINF, l))

        mreg = lax.fori_loop(0, nit, pA, jnp.full((16,), NINF, jnp.float32))
        mx_loc = jnp.max(mreg, axis=0)
        mxv = jnp.broadcast_to(mx_loc, (16,))

        def pB(i, carry):
            cp, se, bv, bn, bl = carry
            l = lg_v[pl.ds(i * 16, 16)]
            c = cnt_v[pl.ds(i * 16, 16)]
            validb = c == 0
            vi = jnp.where(validb, 1, 0).astype(jnp.int32)
            incl = plsc.cumsum(vi)
            pos = cp + incl - vi
            gn = plsc.load_gather(gn_v, [pos])
            val = jnp.where(validb, l + gn, NINF)
            upd = val > bv
            bv = jnp.where(upd, val, bv)
            bn = jnp.where(upd, lanes + base + i * 16, bn)
            bl = jnp.where(upd, l, bl)
            se = se + jnp.where(validb, jnp.exp(l - mxv), jnp.float32(0.0))
            cp = cp + plsc.all_reduce_population_count(validb)
            return (cp, se, bv, bn, bl)

        z16i = jnp.zeros((16,), jnp.int32)
        cp, se, bv, bn, bl = lax.fori_loop(
            0, nit, pB,
            (off, jnp.zeros((16,), jnp.float32), jnp.full((16,), NINF, jnp.float32),
             z16i, jnp.zeros((16,), jnp.float32)))

        M_loc = jnp.max(bv, axis=0)
        eqv = bv == jnp.broadcast_to(M_loc, (16,))
        wn_loc = jnp.min(jnp.where(eqv, bn, IMAX), axis=0)
        wnv_loc = jnp.broadcast_to(wn_loc, (16,))
        bl_loc = jnp.max(jnp.where(jnp.logical_and(eqv, bn == wnv_loc), bl, NINF),
                         axis=0)
        S_loc = jnp.sum(se, axis=0)
        fpub[...] = jnp.where(lanes == 0, mxv,
                     jnp.where(lanes == 1, jnp.broadcast_to(S_loc, (16,)),
                      jnp.where(lanes == 2, jnp.broadcast_to(M_loc, (16,)),
                                jnp.broadcast_to(bl_loc, (16,)))))
        pltpu.sync_copy(fpub, fres_sh.at[sid])
        i16b[...] = wnv_loc
        pltpu.sync_copy(i16b, ires_sh.at[sid])

    plsc.subcore_barrier()

    @pl.when(jnp.logical_and(on0, sid == 0))
    def _():
        pltpu.sync_copy(fres_sh, f256_v)
        pltpu.sync_copy(ires_sh, c256_v)
        z16 = jnp.zeros((16,), jnp.int32)
        mx_t = plsc.load_gather(f256_v, [lanes, z16])
        S_t = plsc.load_gather(f256_v, [lanes, z16 + 1])
        M_t = plsc.load_gather(f256_v, [lanes, z16 + 2])
        bl_t = plsc.load_gather(f256_v, [lanes, z16 + 3])
        wn_t = plsc.load_gather(c256_v, [lanes, z16])
        mx = jnp.max(mx_t, axis=0)
        mxv = jnp.broadcast_to(mx, (16,))
        S = jnp.sum(S_t * jnp.exp(mx_t - mxv), axis=0)
        M = jnp.max(M_t, axis=0)
        eqt = M_t == jnp.broadcast_to(M, (16,))
        wn = jnp.min(jnp.where(eqt, wn_t, IMAX), axis=0)
        wnv = jnp.broadcast_to(wn, (16,))
        blw = jnp.max(jnp.where(jnp.logical_and(eqt, wn_t == wnv), bl_t, NINF),
                      axis=0)
        resv_i[...] = wnv
        resv_f[...] = jnp.where(lanes == 0, jnp.broadcast_to(blw, (16,)),
                                jnp.where(lanes == 1, mxv,
                                          jnp.broadcast_to(S, (16,))))
        pltpu.sync_copy(resv_i, resi_hbm)
        pltpu.sync_copy(resv_f, resf_hbm)


def _sc3(src, dst, vf16, lg, gn):
    k = pl.kernel(
        _sc3_body,
        out_type=(jax.ShapeDtypeStruct((16,), jnp.int32),
                  jax.ShapeDtypeStruct((16,), jnp.float32)),
        mesh=_mesh(),
        compiler_params=pltpu.CompilerParams(needs_layout_passes=False,
                                             use_tc_tiling_on_sc=False),
        scratch_types=[
            pltpu.VMEM((E // NS,), jnp.int32),
            pltpu.VMEM((E // NS,), jnp.int32),
            pltpu.VMEM((E // NS,), jnp.int32),
            pltpu.VMEM((640,), jnp.int32),
            pltpu.VMEM((640,), jnp.float32),
            pltpu.VMEM((648,), jnp.float32),
            pltpu.VMEM((16,), jnp.int32),
            pltpu.VMEM((16,), jnp.int32),
            pltpu.VMEM((16,), jnp.int32),
            pltpu.VMEM((16,), jnp.int32),
            pltpu.VMEM((16,), jnp.float32),
            pltpu.VMEM((16,), jnp.float32),
            pltpu.VMEM((16, 16), jnp.int32),
            pltpu.VMEM((16, 16), jnp.float32),
            pltpu.VMEM_SHARED((N,), jnp.int32),
            pltpu.VMEM_SHARED((16, 16), jnp.int32),
            pltpu.VMEM_SHARED((16, 16), jnp.float32),
            pltpu.VMEM_SHARED((16, 16), jnp.int32),
            pltpu.SemaphoreType.DMA,
        ],
    )
    return k(src, dst, vf16, lg, gn)


def kernel(x, edge_index, W_gnn, b_gnn, W1e, b1e, W2e, b2e, W1t, b1t, W2t, b2t):
    src = edge_index[0]
    dst = edge_index[1]

    noiseE = jax.random.gumbel(jax.random.key(42), (E,), jnp.float32)
    gnoise = jax.random.gumbel(jax.random.key(43), (N,), jnp.float32)

    agg2 = _sc1(edge_index, x)

    emb, nodeH, nodeT, gsum, noise8 = _tc1(x, agg2, W_gnn, b_gnn[None, :],
                                           W1e[D:], W1t[3 * D:],
                                           noiseE.reshape(E8, 8))

    hs_s, hs_d = _sc2(src, dst, nodeH)

    eye8 = jnp.eye(8, dtype=jnp.float32)
    W1e_top_t = jnp.tile(W1e[:D], (1, 8))
    b1e_t = jnp.tile(b1e, 8)[None, :]
    Me = jnp.kron(eye8, W2e)
    b2e_t = jnp.full((1, 8), b2e[0], jnp.float32)

    eidx, vfir, vsec, lpe = _tc2(
        hs_s.reshape(E8, D), hs_d.reshape(E8, D), noise8,
        src.reshape(E8, 8), dst.reshape(E8, 8), gsum,
        W1e_top_t, b1e_t, Me, b2e_t)

    Wg_t = jnp.tile(W1t[:D], (1, 8))
    Ws_t = jnp.tile(W1t[D:2 * D], (1, 8))
    Wf_t = jnp.tile(W1t[2 * D:3 * D], (1, 8))
    b1t_t = jnp.tile(b1t, 8)[None, :]
    Mt = jnp.kron(eye8, W2t)
    b2t_t = jnp.full((1, 8), b2t[0], jnp.float32)

    tl8 = _tc3(nodeT.reshape(N // 8, D), emb, gsum,
               vfir.reshape(1), vsec.reshape(1),
               Wg_t, Ws_t, Wf_t, b1t_t, Mt, b2t_t)

    vf16 = jnp.broadcast_to(vfir.reshape(1), (16,)).astype(jnp.int32)
    gn_pad = jnp.concatenate([gnoise, jnp.zeros((656,), jnp.float32)])
    resi, resf = _sc3(src, dst, vf16, tl8.reshape(N), gn_pad)

    v_thi = resi[0]
    lp3 = (resf[0] - resf[1]) - jnp.log(resf[2])

    action = jnp.stack([vfir[0].astype(jnp.int32),
                        vsec[0].astype(jnp.int32),
                        v_thi.astype(jnp.int32)])
    return action, (lpe[0] + lp3).astype(jnp.float32)
